# Initial kernel scaffold; baseline (speedup 1.0000x reference)
import jax
import jax.numpy as jnp
from jax.experimental import pallas as pl


def kernel(tables, inputs):
    raise NotImplementedError("write your pallas kernel here")


# speedup vs baseline: 6.3220x; 6.3220x over previous
import functools

import jax
import jax.numpy as jnp
from jax import lax
from jax.experimental import pallas as pl
from jax.experimental.pallas import tpu as pltpu
from jax.experimental.pallas import tpu_sc as plsc

NUM_TABLES = 26
VOCAB = 100000
EMBED_DIM = 32
BATCH = 4096
HOTNESS = 20

_L = 16
_NC = 2
_NS = 16
_NW = _NC * _NS
_BPW = BATCH // _NW
_ROWS_PER_CHUNK = 4
_IDX_PER_CHUNK = _ROWS_PER_CHUNK * HOTNESS
_NCHUNK = _BPW // _ROWS_PER_CHUNK
_IPT = _BPW * HOTNESS
_NBUF = 2


def _tree_sum(vals):
    while len(vals) > 1:
        nxt = [vals[i] + vals[i + 1] for i in range(0, len(vals) - 1, 2)]
        if len(vals) % 2:
            nxt.append(vals[-1])
        vals = nxt
    return vals[0]


def _body(tables_hbm, idx_hbm, out_hbm, idx_raw, io0, io1, st0, st1, acc,
          sem0, sem1):
    wid = lax.axis_index("s") * _NC + lax.axis_index("c")
    b0 = wid * _BPW
    idx_bufs = (io0, io1)
    stg_bufs = (st0, st1)
    sems = (sem0, sem1)

    def prep_and_fire(t_off, c, b):
        for i in range(_IDX_PER_CHUNK // _L):
            src = idx_raw[pl.ds(pl.multiple_of(c * _IDX_PER_CHUNK, 8) + i * _L,
                                _L)]
            idx_bufs[b][pl.ds(i * _L, _L)] = src + t_off
        pltpu.async_copy(tables_hbm.at[idx_bufs[b]], stg_bufs[b], sems[b])

    def wait_gather(b):
        pltpu.make_async_copy(tables_hbm.at[pl.ds(0, _IDX_PER_CHUNK)],
                              stg_bufs[b], sems[b]).wait()

    def accum(t, c, b):
        sref = stg_bufs[b]
        for p in range(_ROWS_PER_CHUNK):
            bl = c * _ROWS_PER_CHUNK + p
            for col in (0, _L):
                vals = [sref[p * HOTNESS + h, pl.ds(col, _L)]
                        for h in range(HOTNESS)]
                acc[bl, t, pl.ds(col, _L)] = _tree_sum(vals)

    def table_body(t, carry):
        t_off = t * VOCAB
        pltpu.sync_copy(
            idx_hbm.at[t, pl.ds(pl.multiple_of(b0 * HOTNESS, 8), _IPT)],
            idx_raw)
        for b in range(_NBUF):
            prep_and_fire(t_off, b, b)

        def chunk_body(cc, inner):
            for b in range(_NBUF):
                c = cc * _NBUF + b
                wait_gather(b)
                accum(t, c, b)

                @pl.when(c + _NBUF < _NCHUNK)
                def _():
                    prep_and_fire(t_off, c + _NBUF, b)
            return inner

        lax.fori_loop(0, _NCHUNK // _NBUF, chunk_body, 0)
        return carry

    lax.fori_loop(0, NUM_TABLES, table_body, 0)
    pltpu.sync_copy(acc, out_hbm.at[pl.ds(pl.multiple_of(b0, 8), _BPW)])


@functools.partial(jax.jit, static_argnums=())
def _run(tables_flat, idx):
    mesh = plsc.VectorSubcoreMesh(core_axis_name="c", subcore_axis_name="s")
    fn = pl.kernel(
        _body,
        out_type=jax.ShapeDtypeStruct((BATCH, NUM_TABLES, EMBED_DIM),
                                      jnp.float32),
        mesh=mesh,
        scratch_types=[
            pltpu.VMEM((_IPT,), jnp.int32),
            pltpu.VMEM((_IDX_PER_CHUNK,), jnp.int32),
            pltpu.VMEM((_IDX_PER_CHUNK,), jnp.int32),
            pltpu.VMEM((_IDX_PER_CHUNK, EMBED_DIM), jnp.float32),
            pltpu.VMEM((_IDX_PER_CHUNK, EMBED_DIM), jnp.float32),
            pltpu.VMEM((_BPW, NUM_TABLES, EMBED_DIM), jnp.float32),
            pltpu.SemaphoreType.DMA,
            pltpu.SemaphoreType.DMA,
        ],
        compiler_params=pltpu.CompilerParams(use_tc_tiling_on_sc=False),
    )
    return fn(tables_flat, idx)


def kernel(tables, inputs):
    tables_flat = tables.reshape(NUM_TABLES * VOCAB, EMBED_DIM)
    idx = inputs.astype(jnp.int32).reshape(NUM_TABLES, BATCH * HOTNESS)
    return _run(tables_flat, idx)
